# baseline (speedup 1.0000x reference)
import jax
import jax.numpy as jnp
from jax import lax
from jax.experimental import pallas as pl
from jax.experimental.pallas import tpu as pltpu
from jax.experimental.pallas import tpu_sc as plsc

L_COORD, L_OBJ, L_NOOBJ = 5.0, 1.0, 0.5
NCORE, NSUB, LANES = 2, 16, 16
NW = NCORE * NSUB
QROWS = 8


def _sc_body(shapes):
    B, BB, H, W = shapes
    BPW = B // NW
    NQ = H // QROWS

    def body(pr, pb, lr, lb, out,
             lrb0, lrb1, prb0, prb1, pbb0, pbb1, lbb0, lbb1, outv, sb0, sb1):
        wid = lax.axis_index("c") * NSUB + lax.axis_index("s")
        m3 = jnp.where(lax.broadcasted_iota(jnp.int32, (LANES,), 0) >= 8,
                       1.0, 0.0).astype(jnp.float32)

        def iou(tx1, ty1, tx2, ty2, qx1, qy1, qx2, qy2):
            ix1 = jnp.maximum(tx1, qx1)
            iy1 = jnp.maximum(ty1, qy1)
            ix2 = jnp.minimum(tx2, qx2)
            iy2 = jnp.minimum(ty2, qy2)
            inter = jnp.maximum(ix2 - ix1, 0.0) * jnp.maximum(iy2 - iy1, 0.0)
            a1 = (tx2 - tx1) * (ty2 - ty1)
            a2 = (qx2 - qx1) * (qy2 - qy1)
            return inter / (a1 + a2 - inter + 0.0001)

        def corners(x, y, w, h):
            x1 = x - w * 0.5
            y1 = y - h * 0.5
            return x1, y1, x1 + w, y1 + h

        zeros = jnp.zeros((LANES,), jnp.float32)
        neg, pobj, off = zeros, zeros, zeros
        bbufs = ((lrb0, prb0, pbb0, lbb0, sb0), (lrb1, prb1, pbb1, lbb1, sb1))

        for bi in range(BPW):
            b = wid * BPW + bi

            def issue_box(q, slot):
                blr, bpr, bp, bl, sb = bbufs[slot]
                r0 = q * QROWS
                pltpu.async_copy(lr.at[b, :, pl.ds(r0, QROWS), :], blr, sb)
                pltpu.async_copy(pr.at[b, :, pl.ds(r0, QROWS), :], bpr, sb)
                pltpu.async_copy(pb.at[b, :, pl.ds(r0, QROWS), :], bp, sb)
                pltpu.async_copy(lb.at[b, :, pl.ds(r0, QROWS), :], bl, sb)

            def wait_box(slot):
                blr, bpr, bp, bl, sb = bbufs[slot]
                pltpu.make_async_copy(lr.at[b, :, pl.ds(0, QROWS), :], blr, sb).wait()
                pltpu.make_async_copy(pr.at[b, :, pl.ds(0, QROWS), :], bpr, sb).wait()
                pltpu.make_async_copy(pb.at[b, :, pl.ds(0, QROWS), :], bp, sb).wait()
                pltpu.make_async_copy(lb.at[b, :, pl.ds(0, QROWS), :], bl, sb).wait()

            issue_box(0, 0)

            for q in range(NQ):
                slot = q % 2
                if q + 1 < NQ:
                    issue_box(q + 1, 1 - slot)
                wait_box(slot)
                lrb, prb, pbb, lbb = (bbufs[slot][0], bbufs[slot][1],
                                      bbufs[slot][2], bbufs[slot][3])

                def box_body(t, carry, lrb=lrb, prb=prb, pbb=pbb, lbb=lbb):
                    neg, pobj, off = carry
                    y = lax.shift_right_logical(t, 2)
                    j = lax.bitwise_and(t, 3)
                    is_tail = j == 3
                    xoff = jnp.where(is_tail, 40, j * LANES)
                    s = pl.ds(xoff, LANES)
                    mj = jnp.where(is_tail, m3, 1.0).astype(jnp.float32)
                    lr0 = lrb[0, y, s]
                    lr1 = lrb[1, y, s]
                    pr0 = prb[0, y, s]
                    pr1 = prb[1, y, s]
                    valid = jnp.where(lr0 + lr1 > 0.9, mj, 0.0)
                    neg = (neg
                           + (pr0 - lr0) * (pr0 - lr0) * jnp.where(lr0 < 1.0, mj, 0.0)
                           + (pr1 - lr1) * (pr1 - lr1) * jnp.where(lr1 < 1.0, mj, 0.0))
                    lx0, ly0, lw0, lh0 = lbb[0, y, s], lbb[1, y, s], lbb[2, y, s], lbb[3, y, s]
                    lx1, ly1, lw1, lh1 = lbb[4, y, s], lbb[5, y, s], lbb[6, y, s], lbb[7, y, s]
                    px0, py0, pw0, ph0 = pbb[0, y, s], pbb[1, y, s], pbb[2, y, s], pbb[3, y, s]
                    px1, py1, pw1, ph1 = pbb[4, y, s], pbb[5, y, s], pbb[6, y, s], pbb[7, y, s]
                    iou0 = iou(*corners(lx0, ly0, lw0, lh0), *corners(px0, py0, pw0, ph0))
                    iou1 = iou(*corners(lx1, ly1, lw1, lh1), *corners(px1, py1, pw1, ph1))
                    sel = iou1 > iou0
                    best_iou = jnp.where(sel, iou1, iou0)
                    best_pr = jnp.where(sel, pr1, pr0)
                    dr = best_pr - best_iou
                    pobj = pobj + dr * dr * valid
                    ex0, ey0, ew0, eh0 = px0 - lx0, py0 - ly0, pw0 - lw0, ph0 - lh0
                    ex1, ey1, ew1, eh1 = px1 - lx1, py1 - ly1, pw1 - lw1, ph1 - lh1
                    off0 = ex0 * ex0 + ey0 * ey0 + ew0 * ew0 + eh0 * eh0
                    off1 = ex1 * ex1 + ey1 * ey1 + ew1 * ew1 + eh1 * eh1
                    off = off + jnp.where(sel, off1, off0) * valid
                    return neg, pobj, off

                neg, pobj, off = lax.fori_loop(0, QROWS * 4, box_body,
                                               (neg, pobj, off))

        outv[0, :] = pobj
        outv[1, :] = neg
        outv[2, :] = off
        pltpu.sync_copy(outv, out.at[wid])

    return body


def _sc_losses(pred_response, pred_bboxes, label_response, label_bboxes):
    B, BB, H, W = pred_response.shape
    mesh = plsc.VectorSubcoreMesh(core_axis_name="c", subcore_axis_name="s")
    f32 = jnp.float32
    run = pl.kernel(
        _sc_body((B, BB, H, W)),
        out_type=jax.ShapeDtypeStruct((NW, 3, LANES), f32),
        mesh=mesh,
        scratch_types=[
            pltpu.VMEM((BB, QROWS, W), f32),
            pltpu.VMEM((BB, QROWS, W), f32),
            pltpu.VMEM((BB, QROWS, W), f32),
            pltpu.VMEM((BB, QROWS, W), f32),
            pltpu.VMEM((BB * 4, QROWS, W), f32),
            pltpu.VMEM((BB * 4, QROWS, W), f32),
            pltpu.VMEM((BB * 4, QROWS, W), f32),
            pltpu.VMEM((BB * 4, QROWS, W), f32),
            pltpu.VMEM((3, LANES), f32),
            pltpu.SemaphoreType.DMA,
            pltpu.SemaphoreType.DMA,
        ],
    )
    return run(pred_response, pred_bboxes, label_response, label_bboxes)


def _tc_body(pc, lc, lr, out_ref):
    b = pl.program_id(0)
    cls_p = 0.0
    for i in range(pc.shape[0]):
        valid = (lr[i, 0:1] + lr[i, 1:2] > 0.9).astype(jnp.float32)
        cls_p += jnp.sum(((pc[i] - lc[i]) ** 2) * valid)
    part = jnp.full((1, 128), cls_p, jnp.float32)

    @pl.when(b == 0)
    def _():
        out_ref[...] = jnp.zeros_like(out_ref)

    out_ref[...] += part


def kernel(pred_cls, pred_response, pred_bboxes, label_cls, label_response, label_bboxes):
    B, CLS, H, W = pred_cls.shape
    BB = pred_response.shape[1]
    HW = H * W

    sc_acc = _sc_losses(pred_response, pred_bboxes, label_response, label_bboxes)

    pc = pred_cls.reshape(B, CLS, HW)
    lc = label_cls.reshape(B, CLS, HW)
    lr = label_response.reshape(B, BB, HW)
    BPG = 8
    cls_acc = pl.pallas_call(
        _tc_body,
        grid=(B // BPG,),
        in_specs=[
            pl.BlockSpec((BPG, CLS, HW), lambda b: (b, 0, 0)),
            pl.BlockSpec((BPG, CLS, HW), lambda b: (b, 0, 0)),
            pl.BlockSpec((BPG, BB, HW), lambda b: (b, 0, 0)),
        ],
        out_specs=pl.BlockSpec((1, 128), lambda b: (0, 0)),
        out_shape=jax.ShapeDtypeStruct((1, 128), jnp.float32),
    )(pc, lc, lr)

    sums = jnp.sum(sc_acc, axis=(0, 2))
    inv_b = 1.0 / B
    return {"pObj": sums[0] * (inv_b * L_OBJ),
            "nObj": sums[1] * (inv_b * L_NOOBJ),
            "cls": cls_acc[0, 0] * inv_b,
            "offset": sums[2] * (inv_b * L_COORD)}


# speedup vs baseline: 1.0027x; 1.0027x over previous
import jax
import jax.numpy as jnp
from jax import lax
from jax.experimental import pallas as pl
from jax.experimental.pallas import tpu as pltpu
from jax.experimental.pallas import tpu_sc as plsc

L_COORD, L_OBJ, L_NOOBJ = 5.0, 1.0, 0.5
NCORE, NSUB, LANES = 2, 16, 16
NW = NCORE * NSUB
QROWS = 8


def _sc_body(shapes):
    B, BB, H, W = shapes
    BPW = B // NW
    NQ = H // QROWS

    def body(pr, pb, lr, lb, out,
             lrb0, lrb1, prb0, prb1, pbb0, pbb1, lbb0, lbb1, outv, sb0, sb1):
        wid = lax.axis_index("c") * NSUB + lax.axis_index("s")
        m3 = jnp.where(lax.broadcasted_iota(jnp.int32, (LANES,), 0) >= 8,
                       1.0, 0.0).astype(jnp.float32)

        def iou(tx1, ty1, tx2, ty2, qx1, qy1, qx2, qy2):
            ix1 = jnp.maximum(tx1, qx1)
            iy1 = jnp.maximum(ty1, qy1)
            ix2 = jnp.minimum(tx2, qx2)
            iy2 = jnp.minimum(ty2, qy2)
            inter = jnp.maximum(ix2 - ix1, 0.0) * jnp.maximum(iy2 - iy1, 0.0)
            a1 = (tx2 - tx1) * (ty2 - ty1)
            a2 = (qx2 - qx1) * (qy2 - qy1)
            return inter / (a1 + a2 - inter + 0.0001)

        def corners(x, y, w, h):
            x1 = x - w * 0.5
            y1 = y - h * 0.5
            return x1, y1, x1 + w, y1 + h

        zeros = jnp.zeros((LANES,), jnp.float32)
        neg, pobj, off = zeros, zeros, zeros
        bbufs = ((lrb0, prb0, pbb0, lbb0, sb0), (lrb1, prb1, pbb1, lbb1, sb1))

        for bi in range(BPW):
            b = wid * BPW + bi

            def issue_box(q, slot):
                blr, bpr, bp, bl, sb = bbufs[slot]
                r0 = q * QROWS
                pltpu.async_copy(lr.at[b, :, pl.ds(r0, QROWS), :], blr, sb)
                pltpu.async_copy(pr.at[b, :, pl.ds(r0, QROWS), :], bpr, sb)
                pltpu.async_copy(pb.at[b, :, pl.ds(r0, QROWS), :], bp, sb)
                pltpu.async_copy(lb.at[b, :, pl.ds(r0, QROWS), :], bl, sb)

            def wait_box(slot):
                blr, bpr, bp, bl, sb = bbufs[slot]
                pltpu.make_async_copy(lr.at[b, :, pl.ds(0, QROWS), :], blr, sb).wait()
                pltpu.make_async_copy(pr.at[b, :, pl.ds(0, QROWS), :], bpr, sb).wait()
                pltpu.make_async_copy(pb.at[b, :, pl.ds(0, QROWS), :], bp, sb).wait()
                pltpu.make_async_copy(lb.at[b, :, pl.ds(0, QROWS), :], bl, sb).wait()

            issue_box(0, 0)

            for q in range(NQ):
                slot = q % 2
                if q + 1 < NQ:
                    issue_box(q + 1, 1 - slot)
                wait_box(slot)
                lrb, prb, pbb, lbb = (bbufs[slot][0], bbufs[slot][1],
                                      bbufs[slot][2], bbufs[slot][3])

                def box_body(t, carry, lrb=lrb, prb=prb, pbb=pbb, lbb=lbb):
                    neg, pobj, off = carry
                    y = lax.shift_right_logical(t, 2)
                    j = lax.bitwise_and(t, 3)
                    is_tail = j == 3
                    xoff = jnp.where(is_tail, 40, j * LANES)
                    s = pl.ds(xoff, LANES)
                    mj = jnp.where(is_tail, m3, 1.0).astype(jnp.float32)
                    lr0 = lrb[0, y, s]
                    lr1 = lrb[1, y, s]
                    pr0 = prb[0, y, s]
                    pr1 = prb[1, y, s]
                    valid = jnp.where(lr0 + lr1 > 0.9, mj, 0.0)
                    neg = (neg
                           + (pr0 - lr0) * (pr0 - lr0) * jnp.where(lr0 < 1.0, mj, 0.0)
                           + (pr1 - lr1) * (pr1 - lr1) * jnp.where(lr1 < 1.0, mj, 0.0))
                    lx0, ly0, lw0, lh0 = lbb[0, y, s], lbb[1, y, s], lbb[2, y, s], lbb[3, y, s]
                    lx1, ly1, lw1, lh1 = lbb[4, y, s], lbb[5, y, s], lbb[6, y, s], lbb[7, y, s]
                    px0, py0, pw0, ph0 = pbb[0, y, s], pbb[1, y, s], pbb[2, y, s], pbb[3, y, s]
                    px1, py1, pw1, ph1 = pbb[4, y, s], pbb[5, y, s], pbb[6, y, s], pbb[7, y, s]
                    iou0 = iou(*corners(lx0, ly0, lw0, lh0), *corners(px0, py0, pw0, ph0))
                    iou1 = iou(*corners(lx1, ly1, lw1, lh1), *corners(px1, py1, pw1, ph1))
                    sel = iou1 > iou0
                    best_iou = jnp.where(sel, iou1, iou0)
                    best_pr = jnp.where(sel, pr1, pr0)
                    dr = best_pr - best_iou
                    pobj = pobj + dr * dr * valid
                    ex0, ey0, ew0, eh0 = px0 - lx0, py0 - ly0, pw0 - lw0, ph0 - lh0
                    ex1, ey1, ew1, eh1 = px1 - lx1, py1 - ly1, pw1 - lw1, ph1 - lh1
                    off0 = ex0 * ex0 + ey0 * ey0 + ew0 * ew0 + eh0 * eh0
                    off1 = ex1 * ex1 + ey1 * ey1 + ew1 * ew1 + eh1 * eh1
                    off = off + jnp.where(sel, off1, off0) * valid
                    return neg, pobj, off

                neg, pobj, off = lax.fori_loop(0, QROWS * 4, box_body,
                                               (neg, pobj, off))

        outv[0, :] = pobj
        outv[1, :] = neg
        outv[2, :] = off
        pltpu.sync_copy(outv, out.at[wid])

    return body


def _sc_losses(pred_response, pred_bboxes, label_response, label_bboxes):
    B, BB, H, W = pred_response.shape
    mesh = plsc.VectorSubcoreMesh(core_axis_name="c", subcore_axis_name="s")
    f32 = jnp.float32
    run = pl.kernel(
        _sc_body((B, BB, H, W)),
        out_type=jax.ShapeDtypeStruct((NW, 3, LANES), f32),
        mesh=mesh,
        scratch_types=[
            pltpu.VMEM((BB, QROWS, W), f32),
            pltpu.VMEM((BB, QROWS, W), f32),
            pltpu.VMEM((BB, QROWS, W), f32),
            pltpu.VMEM((BB, QROWS, W), f32),
            pltpu.VMEM((BB * 4, QROWS, W), f32),
            pltpu.VMEM((BB * 4, QROWS, W), f32),
            pltpu.VMEM((BB * 4, QROWS, W), f32),
            pltpu.VMEM((BB * 4, QROWS, W), f32),
            pltpu.VMEM((3, LANES), f32),
            pltpu.SemaphoreType.DMA,
            pltpu.SemaphoreType.DMA,
        ],
    )
    return run(pred_response, pred_bboxes, label_response, label_bboxes)


def _tc_body(pc, lc, lr, out_ref):
    b = pl.program_id(0)
    cls_p = 0.0
    for i in range(pc.shape[0]):
        valid = (lr[i, 0:1] + lr[i, 1:2] > 0.9).astype(jnp.float32)
        cls_p += jnp.sum(((pc[i] - lc[i]) ** 2) * valid)
    part = jnp.full((1, 128), cls_p, jnp.float32)

    @pl.when(b == 0)
    def _():
        out_ref[...] = jnp.zeros_like(out_ref)

    out_ref[...] += part


def kernel(pred_cls, pred_response, pred_bboxes, label_cls, label_response, label_bboxes):
    B, CLS, H, W = pred_cls.shape
    BB = pred_response.shape[1]
    HW = H * W

    sc_acc = _sc_losses(pred_response, pred_bboxes, label_response, label_bboxes)

    pc = pred_cls.reshape(B, CLS, HW)
    lc = label_cls.reshape(B, CLS, HW)
    lr = label_response.reshape(B, BB, HW)
    BPG = 4
    cls_acc = pl.pallas_call(
        _tc_body,
        grid=(B // BPG,),
        in_specs=[
            pl.BlockSpec((BPG, CLS, HW), lambda b: (b, 0, 0)),
            pl.BlockSpec((BPG, CLS, HW), lambda b: (b, 0, 0)),
            pl.BlockSpec((BPG, BB, HW), lambda b: (b, 0, 0)),
        ],
        out_specs=pl.BlockSpec((1, 128), lambda b: (0, 0)),
        out_shape=jax.ShapeDtypeStruct((1, 128), jnp.float32),
    )(pc, lc, lr)

    sums = jnp.sum(sc_acc, axis=(0, 2))
    inv_b = 1.0 / B
    return {"pObj": sums[0] * (inv_b * L_OBJ),
            "nObj": sums[1] * (inv_b * L_NOOBJ),
            "cls": cls_acc[0, 0] * inv_b,
            "offset": sums[2] * (inv_b * L_COORD)}
